# Initial kernel scaffold; baseline (speedup 1.0000x reference)
import jax
import jax.numpy as jnp
from jax.experimental import pallas as pl


def kernel(x, edge_index, lin1_W, lin1_b, blk_g, blk_b, conv_Wl, conv_bl, conv_Wr, codebooks, fin_g, fin_b, lin2_W, lin2_b):
    raise NotImplementedError("write your pallas kernel here")


# speedup vs baseline: 3.0094x; 3.0094x over previous
import functools

import jax
import jax.numpy as jnp
from jax import lax
from jax.experimental import pallas as pl
from jax.experimental.pallas import tpu as pltpu
from jax.experimental.pallas import tpu_sc as plsc

N = 10000
E = 160000
H = 256
HH = 128
L = 3
G = 3
K = 16
KP = 128
EPS = 1e-5
R = 2000
NBLK = N // R

NSUB = 16
EPT = E // NSUB
ECH = 80
NCH = EPT // ECH
EPT_D = E // (2 * NSUB)
DCH = 40
NCH_D = EPT_D // DCH
ACCN = 10240
ROWS_PT = ACCN // NSUB

def _dot(a, b, dims, prec=jax.lax.Precision.DEFAULT):
  return lax.dot_general(a, b, dimension_numbers=(dims, ((), ())),
                         preferred_element_type=jnp.float32, precision=prec)


def _ln_relu(x, g, b):
  m = jnp.mean(x, axis=-1, keepdims=True)
  d = x - m
  v = jnp.mean(d * d, axis=-1, keepdims=True)
  return jnp.maximum(d * lax.rsqrt(v + EPS) * g + b, 0.0)


def _pre_body(x_ref, w1t_ref, b1_ref, g0_ref, bb0_ref, h_ref):
  x1 = _dot(x_ref[...], w1t_ref[...], ((1,), (0,))) + b1_ref[...]
  h = _ln_relu(x1, g0_ref[...], bb0_ref[...])
  h_ref[0] = h[:, :HH]
  h_ref[1] = h[:, HH:]


def _layer_body(last, h_ref, a_ref, degp_ref, wlt_ref, wrt_ref, bl_ref,
                cb_ref, cbsq_ref, gn_ref, bn_ref, w2t_ref, b2_ref,
                o_ref, ids_ref, lp_ref):
  h = jnp.concatenate([h_ref[0], h_ref[1]], axis=-1)
  a = jnp.concatenate([a_ref[0], a_ref[1]], axis=-1)
  deg = degp_ref[0, :, 0:1] + degp_ref[1, :, 0:1]
  mean = a / jnp.maximum(deg, 1.0)
  x = _dot(mean, wlt_ref[...], ((1,), (0,))) + bl_ref[...] \
      + _dot(h, wrt_ref[...], ((1,), (0,)))
  r = x
  lane = lax.broadcasted_iota(jnp.int32, (R, KP), 1)
  ids = jnp.zeros((R, KP), jnp.int32)
  for g in range(G):
    cb = cb_ref[g]
    rsq = jnp.sum(r * r, axis=-1, keepdims=True)
    scores = rsq - 2.0 * _dot(r, cb, ((1,), (1,))) + cbsq_ref[g]
    idx = jnp.argmin(scores, axis=1).astype(jnp.int32)
    onehot = (lane == idx[:, None]).astype(jnp.float32)
    q = _dot(onehot, cb, ((1,), (0,)), prec=jax.lax.Precision.HIGHEST)
    r = r - q
    ids = jnp.where(lane == g, idx[:, None], ids)
  ids_ref[...] = ids
  lp_ref[...] = jnp.broadcast_to(jnp.sum(r * r), (1, 1, KP))
  if last:
    xo = _ln_relu(x, gn_ref[...], bn_ref[...])
    o_ref[...] = _dot(xo, w2t_ref[...], ((1,), (0,))) + b2_ref[...]
  else:
    hn = _ln_relu(x, gn_ref[...], bn_ref[...])
    o_ref[0] = hn[:, :HH]
    o_ref[1] = hn[:, HH:]


def _full(shape):
  nd = len(shape)
  return pl.BlockSpec(shape, lambda i: (0,) * nd)


def _pre_call(x, w1t, b1, g0, bb0):
  return pl.pallas_call(
      _pre_body,
      grid=(NBLK,),
      in_specs=[
          pl.BlockSpec((R, H), lambda i: (i, 0)),
          _full((H, H)), _full((1, H)), _full((1, H)), _full((1, H)),
      ],
      out_specs=pl.BlockSpec((2, R, HH), lambda i: (0, i, 0)),
      out_shape=jax.ShapeDtypeStruct((2, N, HH), jnp.float32),
  )(x, w1t, b1, g0, bb0)


def _layer_call(last, h2, agg, degp, wlt, wrt, bl, cbp, cbsq, gn, bn, w2t, b2):
  if last:
    o_spec = pl.BlockSpec((R, H), lambda i: (i, 0))
    o_shape = jax.ShapeDtypeStruct((N, H), jnp.float32)
  else:
    o_spec = pl.BlockSpec((2, R, HH), lambda i: (0, i, 0))
    o_shape = jax.ShapeDtypeStruct((2, N, HH), jnp.float32)
  return pl.pallas_call(
      functools.partial(_layer_body, last),
      grid=(NBLK,),
      in_specs=[
          pl.BlockSpec((2, R, HH), lambda i: (0, i, 0)),
          pl.BlockSpec((2, R, HH), lambda i: (0, i, 0)),
          pl.BlockSpec((2, R, HH), lambda i: (0, i, 0)),
          _full((H, H)), _full((H, H)), _full((1, H)),
          _full((G, KP, H)), _full((G, 1, KP)),
          _full((1, H)), _full((1, H)), _full((H, H)), _full((1, H)),
      ],
      out_specs=[
          o_spec,
          pl.BlockSpec((R, KP), lambda i: (i, 0)),
          pl.BlockSpec((1, 1, KP), lambda i: (i, 0, 0)),
      ],
      out_shape=[
          o_shape,
          jax.ShapeDtypeStruct((N, KP), jnp.int32),
          jax.ShapeDtypeStruct((NBLK, 1, KP), jnp.float32),
      ],
  )(h2, agg, degp, wlt, wrt, bl, cbp, cbsq, gn, bn, w2t, b2)


def _agg_body(h2, src, dst, zeros, out, sidx, didx, rows, acc, sem):
  c = lax.axis_index("c")
  s = lax.axis_index("s")

  @pl.when(s == 0)
  def _():
    pltpu.sync_copy(zeros, acc)

  plsc.subcore_barrier()
  ebase = s * EPT
  off = c * N

  def body(j, carry):
    eb = ebase + j * ECH
    pltpu.sync_copy(src.at[pl.ds(eb, ECH)], sidx)
    pltpu.sync_copy(dst.at[pl.ds(eb, ECH)], didx)
    for t in range(ECH // 16):
      sl = pl.ds(t * 16, 16)
      sidx[sl] = sidx[sl] + off
    pltpu.async_copy(h2.at[sidx], rows, sem).wait()
    pltpu.sync_copy(rows, acc.at[didx], add=True)
    return carry

  lax.fori_loop(0, NCH, body, 0)
  plsc.subcore_barrier()
  rb = s * ROWS_PT
  pltpu.sync_copy(acc.at[pl.ds(rb, ROWS_PT)],
                  out.at[pl.ds(c * ACCN + rb, ROWS_PT)])


def _deg_body(dst, zeros, ones, out, didx, ones_v, acc):
  c = lax.axis_index("c")
  s = lax.axis_index("s")

  @pl.when(s == 0)
  def _():
    pltpu.sync_copy(zeros, acc)

  pltpu.sync_copy(ones, ones_v)
  plsc.subcore_barrier()
  ebase = c * (E // 2) + s * EPT_D

  def body(j, carry):
    pltpu.sync_copy(dst.at[pl.ds(ebase + j * DCH, DCH)], didx)
    pltpu.sync_copy(ones_v, acc.at[didx], add=True)
    return carry

  lax.fori_loop(0, NCH_D, body, 0)
  plsc.subcore_barrier()
  rb = s * ROWS_PT
  pltpu.sync_copy(acc.at[pl.ds(rb, ROWS_PT)],
                  out.at[pl.ds(c * ACCN + rb, ROWS_PT)])


@functools.cache
def _sc_kernels():
  mesh = plsc.VectorSubcoreMesh(core_axis_name="c", subcore_axis_name="s")
  agg = pl.kernel(
      _agg_body,
      out_type=jax.ShapeDtypeStruct((2 * ACCN, HH), jnp.float32),
      mesh=mesh,
      scratch_types=[
          pltpu.VMEM((ECH,), jnp.int32),
          pltpu.VMEM((ECH,), jnp.int32),
          pltpu.VMEM((ECH, HH), jnp.float32),
          pltpu.VMEM_SHARED((ACCN, HH), jnp.float32),
          pltpu.SemaphoreType.DMA,
      ],
  )
  deg = pl.kernel(
      _deg_body,
      out_type=jax.ShapeDtypeStruct((2 * ACCN, HH), jnp.float32),
      mesh=mesh,
      scratch_types=[
          pltpu.VMEM((DCH,), jnp.int32),
          pltpu.VMEM((DCH, HH), jnp.float32),
          pltpu.VMEM_SHARED((ACCN, HH), jnp.float32),
      ],
  )
  return agg, deg


def kernel(x, edge_index, lin1_W, lin1_b, blk_g, blk_b, conv_Wl, conv_bl,
           conv_Wr, codebooks, fin_g, fin_b, lin2_W, lin2_b):
  src = edge_index[0]
  dst = edge_index[1]

  w1t = lin1_W.T
  w2t = lin2_W.T
  cbp = jnp.pad(codebooks, ((0, 0), (0, 0), (0, KP - K), (0, 0)))
  cbsq = jnp.pad((codebooks * codebooks).sum(-1),
                 ((0, 0), (0, 0), (0, KP - K)),
                 constant_values=1e30)[:, :, None, :]
  zeros128 = jnp.zeros((ACCN, HH), jnp.float32)
  ones128 = jnp.ones((DCH, HH), jnp.float32)

  agg_kernel, deg_kernel = _sc_kernels()
  h2 = _pre_call(x, w1t, lin1_b[None], blk_g[0][None], blk_b[0][None])
  degf = deg_kernel(dst, zeros128, ones128)
  degp = jnp.stack([degf[:N], degf[ACCN:ACCN + N]])

  losses = []
  ids = []
  out = None
  for i in range(L):
    aggf = agg_kernel(h2.reshape(2 * N, HH), src, dst, zeros128)
    agg = jnp.stack([aggf[:N], aggf[ACCN:ACCN + N]])
    last = i == L - 1
    gn = fin_g[None] if last else blk_g[i + 1][None]
    bn = fin_b[None] if last else blk_b[i + 1][None]
    o, ids_i, lp = _layer_call(
        last, h2, agg, degp,
        conv_Wl[i].T, conv_Wr[i].T, conv_bl[i][None],
        cbp[i], cbsq[i], gn, bn, w2t, lin2_b[None])
    if last:
      out = o
    else:
      h2 = o
    losses.append(jnp.sum(lp[:, 0, 0]) / (N * H))
    ids.append(ids_i[:, :G])

  total_loss = losses[0] + losses[1] + losses[2]
  return out, total_loss, jnp.concatenate(ids, axis=1)


# speedup vs baseline: 3.0268x; 1.0058x over previous
import functools

import jax
import jax.numpy as jnp
from jax import lax
from jax.experimental import pallas as pl
from jax.experimental.pallas import tpu as pltpu
from jax.experimental.pallas import tpu_sc as plsc

N = 10000
E = 160000
H = 256
HH = 128
L = 3
G = 3
K = 16
KP = 128
EPS = 1e-5
R = 2000
NBLK = N // R

NSUB = 16
ECH = 128
EP = 163840
EPT = EP // NSUB
NCH = EPT // ECH
EPT_D = EP // (2 * NSUB)
DCH = 128
NCH_D = EPT_D // DCH
ACCN = 10240
ROWS_PT = ACCN // NSUB

def _dot(a, b, dims, prec=jax.lax.Precision.DEFAULT):
  return lax.dot_general(a, b, dimension_numbers=(dims, ((), ())),
                         preferred_element_type=jnp.float32, precision=prec)


def _ln_relu(x, g, b):
  m = jnp.mean(x, axis=-1, keepdims=True)
  d = x - m
  v = jnp.mean(d * d, axis=-1, keepdims=True)
  return jnp.maximum(d * lax.rsqrt(v + EPS) * g + b, 0.0)


def _pre_body(x_ref, w1t_ref, b1_ref, g0_ref, bb0_ref, h_ref):
  x1 = _dot(x_ref[...], w1t_ref[...], ((1,), (0,))) + b1_ref[...]
  h = _ln_relu(x1, g0_ref[...], bb0_ref[...])
  h_ref[0] = h[:, :HH]
  h_ref[1] = h[:, HH:]


def _layer_body(last, h_ref, a_ref, degp_ref, wlt_ref, wrt_ref, bl_ref,
                cb_ref, cbsq_ref, gn_ref, bn_ref, w2t_ref, b2_ref,
                o_ref, ids_ref, lp_ref):
  h = jnp.concatenate([h_ref[0], h_ref[1]], axis=-1)
  a = jnp.concatenate([a_ref[0], a_ref[1]], axis=-1)
  deg = degp_ref[0, :, 0:1] + degp_ref[1, :, 0:1]
  mean = a / jnp.maximum(deg, 1.0)
  x = _dot(mean, wlt_ref[...], ((1,), (0,))) + bl_ref[...] \
      + _dot(h, wrt_ref[...], ((1,), (0,)))
  r = x
  lane = lax.broadcasted_iota(jnp.int32, (R, KP), 1)
  ids = jnp.zeros((R, KP), jnp.int32)
  for g in range(G):
    cb = cb_ref[g]
    rsq = jnp.sum(r * r, axis=-1, keepdims=True)
    scores = rsq - 2.0 * _dot(r, cb, ((1,), (1,))) + cbsq_ref[g]
    idx = jnp.argmin(scores, axis=1).astype(jnp.int32)
    onehot = (lane == idx[:, None]).astype(jnp.float32)
    q = _dot(onehot, cb, ((1,), (0,)), prec=jax.lax.Precision.HIGHEST)
    r = r - q
    ids = jnp.where(lane == g, idx[:, None], ids)
  ids_ref[...] = ids
  lp_ref[...] = jnp.broadcast_to(jnp.sum(r * r), (1, 1, KP))
  if last:
    xo = _ln_relu(x, gn_ref[...], bn_ref[...])
    o_ref[...] = _dot(xo, w2t_ref[...], ((1,), (0,))) + b2_ref[...]
  else:
    hn = _ln_relu(x, gn_ref[...], bn_ref[...])
    o_ref[0] = hn[:, :HH]
    o_ref[1] = hn[:, HH:]


def _full(shape):
  nd = len(shape)
  return pl.BlockSpec(shape, lambda i: (0,) * nd)


def _pre_call(x, w1t, b1, g0, bb0):
  return pl.pallas_call(
      _pre_body,
      grid=(NBLK,),
      in_specs=[
          pl.BlockSpec((R, H), lambda i: (i, 0)),
          _full((H, H)), _full((1, H)), _full((1, H)), _full((1, H)),
      ],
      out_specs=pl.BlockSpec((2, R, HH), lambda i: (0, i, 0)),
      out_shape=jax.ShapeDtypeStruct((2, N, HH), jnp.float32),
  )(x, w1t, b1, g0, bb0)


def _layer_call(last, h2, agg, degp, wlt, wrt, bl, cbp, cbsq, gn, bn, w2t, b2):
  if last:
    o_spec = pl.BlockSpec((R, H), lambda i: (i, 0))
    o_shape = jax.ShapeDtypeStruct((N, H), jnp.float32)
  else:
    o_spec = pl.BlockSpec((2, R, HH), lambda i: (0, i, 0))
    o_shape = jax.ShapeDtypeStruct((2, N, HH), jnp.float32)
  return pl.pallas_call(
      functools.partial(_layer_body, last),
      grid=(NBLK,),
      in_specs=[
          pl.BlockSpec((2, R, HH), lambda i: (0, i, 0)),
          pl.BlockSpec((2, R, HH), lambda i: (0, i, 0)),
          pl.BlockSpec((2, R, HH), lambda i: (0, i, 0)),
          _full((H, H)), _full((H, H)), _full((1, H)),
          _full((G, KP, H)), _full((G, 1, KP)),
          _full((1, H)), _full((1, H)), _full((H, H)), _full((1, H)),
      ],
      out_specs=[
          o_spec,
          pl.BlockSpec((R, KP), lambda i: (i, 0)),
          pl.BlockSpec((1, 1, KP), lambda i: (i, 0, 0)),
      ],
      out_shape=[
          o_shape,
          jax.ShapeDtypeStruct((N, KP), jnp.int32),
          jax.ShapeDtypeStruct((NBLK, 1, KP), jnp.float32),
      ],
  )(h2, agg, degp, wlt, wrt, bl, cbp, cbsq, gn, bn, w2t, b2)


def _agg_body(h2, src2, dst, zeros, out, sidx0, sidx1, didx0, didx1,
              rows0, rows1, acc, sem0, sem1):
  c = lax.axis_index("c")
  s = lax.axis_index("s")

  @pl.when(s == 0)
  def _():
    pltpu.sync_copy(zeros, acc)

  plsc.subcore_barrier()
  sbase = c * EP + s * EPT
  dbase = s * EPT

  def load(j, sidx, didx):
    pltpu.sync_copy(src2.at[pl.ds(sbase + j * ECH, ECH)], sidx)
    pltpu.sync_copy(dst.at[pl.ds(dbase + j * ECH, ECH)], didx)

  load(0, sidx0, didx0)
  pltpu.async_copy(h2.at[sidx0], rows0, sem0)

  def body(i, carry):
    load(2 * i + 1, sidx1, didx1)
    pltpu.async_copy(h2.at[sidx1], rows1, sem1)
    pltpu.make_async_copy(h2.at[sidx0], rows0, sem0).wait()
    pltpu.sync_copy(rows0, acc.at[didx0], add=True)

    @pl.when(i < NCH // 2 - 1)
    def _():
      load(2 * i + 2, sidx0, didx0)
      pltpu.async_copy(h2.at[sidx0], rows0, sem0)

    pltpu.make_async_copy(h2.at[sidx1], rows1, sem1).wait()
    pltpu.sync_copy(rows1, acc.at[didx1], add=True)
    return carry

  lax.fori_loop(0, NCH // 2, body, 0)
  plsc.subcore_barrier()
  rb = s * ROWS_PT
  pltpu.sync_copy(acc.at[pl.ds(rb, ROWS_PT)],
                  out.at[pl.ds(c * ACCN + rb, ROWS_PT)])


def _deg_body(dst, zeros, ones, out, didx, ones_v, acc):
  c = lax.axis_index("c")
  s = lax.axis_index("s")

  @pl.when(s == 0)
  def _():
    pltpu.sync_copy(zeros, acc)

  pltpu.sync_copy(ones, ones_v)
  plsc.subcore_barrier()
  ebase = c * (EP // 2) + s * EPT_D

  def body(j, carry):
    pltpu.sync_copy(dst.at[pl.ds(ebase + j * DCH, DCH)], didx)
    pltpu.sync_copy(ones_v, acc.at[didx], add=True)
    return carry

  lax.fori_loop(0, NCH_D, body, 0)
  plsc.subcore_barrier()
  rb = s * ROWS_PT
  pltpu.sync_copy(acc.at[pl.ds(rb, ROWS_PT)],
                  out.at[pl.ds(c * ACCN + rb, ROWS_PT)])


@functools.cache
def _sc_kernels():
  mesh = plsc.VectorSubcoreMesh(core_axis_name="c", subcore_axis_name="s")
  agg = pl.kernel(
      _agg_body,
      out_type=jax.ShapeDtypeStruct((2 * ACCN, HH), jnp.float32),
      mesh=mesh,
      scratch_types=[
          pltpu.VMEM((ECH,), jnp.int32),
          pltpu.VMEM((ECH,), jnp.int32),
          pltpu.VMEM((ECH,), jnp.int32),
          pltpu.VMEM((ECH,), jnp.int32),
          pltpu.VMEM((ECH, HH), jnp.float32),
          pltpu.VMEM((ECH, HH), jnp.float32),
          pltpu.VMEM_SHARED((ACCN, HH), jnp.float32),
          pltpu.SemaphoreType.DMA,
          pltpu.SemaphoreType.DMA,
      ],
  )
  deg = pl.kernel(
      _deg_body,
      out_type=jax.ShapeDtypeStruct((2 * ACCN, HH), jnp.float32),
      mesh=mesh,
      scratch_types=[
          pltpu.VMEM((DCH,), jnp.int32),
          pltpu.VMEM((DCH, HH), jnp.float32),
          pltpu.VMEM_SHARED((ACCN, HH), jnp.float32),
      ],
  )
  return agg, deg


def kernel(x, edge_index, lin1_W, lin1_b, blk_g, blk_b, conv_Wl, conv_bl,
           conv_Wr, codebooks, fin_g, fin_b, lin2_W, lin2_b):
  srcp = jnp.concatenate([edge_index[0], jnp.zeros((EP - E,), jnp.int32)])
  dst = jnp.concatenate([edge_index[1], jnp.full((EP - E,), N, jnp.int32)])
  src2 = jnp.concatenate([srcp, srcp + N])

  w1t = lin1_W.T
  w2t = lin2_W.T
  cbp = jnp.pad(codebooks, ((0, 0), (0, 0), (0, KP - K), (0, 0)))
  cbsq = jnp.pad((codebooks * codebooks).sum(-1),
                 ((0, 0), (0, 0), (0, KP - K)),
                 constant_values=1e30)[:, :, None, :]
  zeros128 = jnp.zeros((ACCN, HH), jnp.float32)
  ones128 = jnp.ones((DCH, HH), jnp.float32)

  agg_kernel, deg_kernel = _sc_kernels()
  h2 = _pre_call(x, w1t, lin1_b[None], blk_g[0][None], blk_b[0][None])
  degf = deg_kernel(dst, zeros128, ones128)
  degp = jnp.stack([degf[:N], degf[ACCN:ACCN + N]])

  losses = []
  ids = []
  out = None
  for i in range(L):
    aggf = agg_kernel(h2.reshape(2 * N, HH), src2, dst, zeros128)
    agg = jnp.stack([aggf[:N], aggf[ACCN:ACCN + N]])
    last = i == L - 1
    gn = fin_g[None] if last else blk_g[i + 1][None]
    bn = fin_b[None] if last else blk_b[i + 1][None]
    o, ids_i, lp = _layer_call(
        last, h2, agg, degp,
        conv_Wl[i].T, conv_Wr[i].T, conv_bl[i][None],
        cbp[i], cbsq[i], gn, bn, w2t, lin2_b[None])
    if last:
      out = o
    else:
      h2 = o
    losses.append(jnp.sum(lp[:, 0, 0]) / (N * H))
    ids.append(ids_i[:, :G])

  total_loss = losses[0] + losses[1] + losses[2]
  return out, total_loss, jnp.concatenate(ids, axis=1)


# speedup vs baseline: 3.1473x; 1.0398x over previous
import functools

import jax
import jax.numpy as jnp
from jax import lax
from jax.experimental import pallas as pl
from jax.experimental.pallas import tpu as pltpu
from jax.experimental.pallas import tpu_sc as plsc

N = 10000
E = 160000
H = 256
HH = 128
L = 3
G = 3
K = 16
KP = 128
EPS = 1e-5
R = 2000
NBLK = N // R

NSUB = 16
ECH = 80
EP = 163840
EPT = EP // NSUB
NCH = EPT // ECH
IBK = 4
EPT_D = EP // (2 * NSUB)
DCH = 128
NCH_D = EPT_D // DCH
ACCN = 10240
ROWS_PT = ACCN // NSUB

def _dot(a, b, dims, prec=jax.lax.Precision.DEFAULT):
  return lax.dot_general(a, b, dimension_numbers=(dims, ((), ())),
                         preferred_element_type=jnp.float32, precision=prec)


def _ln_relu(x, g, b):
  m = jnp.mean(x, axis=-1, keepdims=True)
  d = x - m
  v = jnp.mean(d * d, axis=-1, keepdims=True)
  return jnp.maximum(d * lax.rsqrt(v + EPS) * g + b, 0.0)


def _pre_body(x_ref, w1t_ref, b1_ref, g0_ref, bb0_ref, h_ref):
  x1 = _dot(x_ref[...], w1t_ref[...], ((1,), (0,))) + b1_ref[...]
  h = _ln_relu(x1, g0_ref[...], bb0_ref[...])
  h_ref[0] = h[:, :HH]
  h_ref[1] = h[:, HH:]


def _layer_body(last, h_ref, a_ref, degp_ref, wlt_ref, wrt_ref, bl_ref,
                cb_ref, cbsq_ref, gn_ref, bn_ref, w2t_ref, b2_ref,
                o_ref, ids_ref, lp_ref):
  h = jnp.concatenate([h_ref[0], h_ref[1]], axis=-1)
  a = jnp.concatenate([a_ref[0], a_ref[1]], axis=-1)
  deg = degp_ref[0, :, 0:1] + degp_ref[1, :, 0:1]
  mean = a / jnp.maximum(deg, 1.0)
  x = _dot(mean, wlt_ref[...], ((1,), (0,))) + bl_ref[...] \
      + _dot(h, wrt_ref[...], ((1,), (0,)))
  r = x
  lane = lax.broadcasted_iota(jnp.int32, (R, KP), 1)
  ids = jnp.zeros((R, KP), jnp.int32)
  for g in range(G):
    cb = cb_ref[g]
    rsq = jnp.sum(r * r, axis=-1, keepdims=True)
    scores = rsq - 2.0 * _dot(r, cb, ((1,), (1,))) + cbsq_ref[g]
    idx = jnp.argmin(scores, axis=1).astype(jnp.int32)
    onehot = (lane == idx[:, None]).astype(jnp.float32)
    q = _dot(onehot, cb, ((1,), (0,)), prec=jax.lax.Precision.HIGHEST)
    r = r - q
    ids = jnp.where(lane == g, idx[:, None], ids)
  ids_ref[...] = ids
  lp_ref[...] = jnp.broadcast_to(jnp.sum(r * r), (1, 1, KP))
  if last:
    xo = _ln_relu(x, gn_ref[...], bn_ref[...])
    o_ref[...] = _dot(xo, w2t_ref[...], ((1,), (0,))) + b2_ref[...]
  else:
    hn = _ln_relu(x, gn_ref[...], bn_ref[...])
    o_ref[0] = hn[:, :HH]
    o_ref[1] = hn[:, HH:]


def _full(shape):
  nd = len(shape)
  return pl.BlockSpec(shape, lambda i: (0,) * nd)


def _pre_call(x, w1t, b1, g0, bb0):
  return pl.pallas_call(
      _pre_body,
      grid=(NBLK,),
      in_specs=[
          pl.BlockSpec((R, H), lambda i: (i, 0)),
          _full((H, H)), _full((1, H)), _full((1, H)), _full((1, H)),
      ],
      out_specs=pl.BlockSpec((2, R, HH), lambda i: (0, i, 0)),
      out_shape=jax.ShapeDtypeStruct((2, N, HH), jnp.float32),
  )(x, w1t, b1, g0, bb0)


def _layer_call(last, h2, agg, degp, wlt, wrt, bl, cbp, cbsq, gn, bn, w2t, b2):
  if last:
    o_spec = pl.BlockSpec((R, H), lambda i: (i, 0))
    o_shape = jax.ShapeDtypeStruct((N, H), jnp.float32)
  else:
    o_spec = pl.BlockSpec((2, R, HH), lambda i: (0, i, 0))
    o_shape = jax.ShapeDtypeStruct((2, N, HH), jnp.float32)
  return pl.pallas_call(
      functools.partial(_layer_body, last),
      grid=(NBLK,),
      in_specs=[
          pl.BlockSpec((2, R, HH), lambda i: (0, i, 0)),
          pl.BlockSpec((2, R, HH), lambda i: (0, i, 0)),
          pl.BlockSpec((2, R, HH), lambda i: (0, i, 0)),
          _full((H, H)), _full((H, H)), _full((1, H)),
          _full((G, KP, H)), _full((G, 1, KP)),
          _full((1, H)), _full((1, H)), _full((H, H)), _full((1, H)),
      ],
      out_specs=[
          o_spec,
          pl.BlockSpec((R, KP), lambda i: (i, 0)),
          pl.BlockSpec((1, 1, KP), lambda i: (i, 0, 0)),
      ],
      out_shape=[
          o_shape,
          jax.ShapeDtypeStruct((N, KP), jnp.int32),
          jax.ShapeDtypeStruct((NBLK, 1, KP), jnp.float32),
      ],
  )(h2, agg, degp, wlt, wrt, bl, cbp, cbsq, gn, bn, w2t, b2)


NBUF = 4


ZR = 32


def _agg_body(h2, src2r, dstr, out, sA, dA, sB, dB,
              rows0, rows1, rows2, rows3, zbuf, acc,
              gs0, gs1, gs2, gs3, ss0, ss1, ss2, ss3):
  c = lax.axis_index("c")
  s = lax.axis_index("s")
  rows = [rows0, rows1, rows2, rows3]
  gsem = [gs0, gs1, gs2, gs3]
  ssem = [ss0, ss1, ss2, ss3]
  nb2 = 2 * IBK

  def zrow(i, carry):
    for t in range(HH // 16):
      zbuf[i, pl.ds(16 * t, 16)] = jnp.zeros((16,), jnp.float32)
    return carry

  lax.fori_loop(0, ZR, zrow, 0)

  def zcopy(j, carry):
    pltpu.sync_copy(zbuf, acc.at[pl.ds(s * ROWS_PT + j * ZR, ZR)])
    return carry

  lax.fori_loop(0, ROWS_PT // ZR, zcopy, 0)
  plsc.subcore_barrier()

  def do_half(bb, sx, dx, sxp, dxp):
    pltpu.sync_copy(
        src2r.at[pl.ds(c * (EP // ECH) + s * NCH + bb * nb2, nb2)], sx)
    pltpu.sync_copy(dstr.at[pl.ds(s * NCH + bb * nb2, nb2)], dx)
    for k in range(nb2):
      j = bb * nb2 + k
      b = k % 4

      @pl.when(j >= 4)
      def _():
        if k >= 4:
          pltpu.make_async_copy(rows[b], acc.at[dx.at[k - 4]], ssem[b]).wait()
        else:
          pltpu.make_async_copy(rows[b], acc.at[dxp.at[k + 4]], ssem[b]).wait()

      pltpu.async_copy(h2.at[sx.at[k]], rows[b], gsem[b])
      b2 = (k - 2) % 4
      s2 = sx.at[k - 2] if k >= 2 else sxp.at[k + nb2 - 2]
      d2 = dx.at[k - 2] if k >= 2 else dxp.at[k + nb2 - 2]

      @pl.when(j >= 2)
      def _():
        pltpu.make_async_copy(h2.at[s2], rows[b2], gsem[b2]).wait()
        pltpu.async_copy(rows[b2], acc.at[d2], ssem[b2], add=True)

  def body(bi, carry):
    do_half(2 * bi, sA, dA, sB, dB)
    do_half(2 * bi + 1, sB, dB, sA, dA)
    return carry

  lax.fori_loop(0, NCH // (2 * nb2), body, 0)
  for k in (nb2 - 2, nb2 - 1):
    b = k % 4
    pltpu.make_async_copy(h2.at[sB.at[k]], rows[b], gsem[b]).wait()
    pltpu.async_copy(rows[b], acc.at[dB.at[k]], ssem[b], add=True)
  for k in range(4, nb2):
    b = k % 4
    pltpu.make_async_copy(rows[b], acc.at[dB.at[k]], ssem[b]).wait()
  plsc.subcore_barrier()
  rb = s * ROWS_PT
  pltpu.sync_copy(acc.at[pl.ds(rb, ROWS_PT)],
                  out.at[pl.ds(c * ACCN + rb, ROWS_PT)])


def _deg_body(dstr, zeros, ones, out, didx_all, ones_v, acc, sem):
  c = lax.axis_index("c")
  s = lax.axis_index("s")

  @pl.when(s == 0)
  def _():
    pltpu.sync_copy(zeros, acc)

  pltpu.sync_copy(ones, ones_v)
  pltpu.sync_copy(
      dstr.at[pl.ds(c * (EP // (2 * DCH)) + s * NCH_D, NCH_D)], didx_all)
  plsc.subcore_barrier()

  def body(j, carry):
    pltpu.async_copy(ones_v, acc.at[didx_all.at[j]], sem, add=True)

    @pl.when(j >= NBUF)
    def _():
      pltpu.make_async_copy(ones_v, acc.at[didx_all.at[j - NBUF]], sem).wait()
    return carry

  lax.fori_loop(0, NCH_D, body, 0)
  for b in range(NBUF):
    pltpu.make_async_copy(ones_v, acc.at[didx_all.at[NCH_D - NBUF + b]],
                          sem).wait()
  plsc.subcore_barrier()
  rb = s * ROWS_PT
  pltpu.sync_copy(acc.at[pl.ds(rb, ROWS_PT)],
                  out.at[pl.ds(c * ACCN + rb, ROWS_PT)])


@functools.cache
def _sc_kernels():
  mesh = plsc.VectorSubcoreMesh(core_axis_name="c", subcore_axis_name="s")
  agg = pl.kernel(
      _agg_body,
      out_type=jax.ShapeDtypeStruct((2 * ACCN, HH), jnp.float32),
      mesh=mesh,
      scratch_types=[
          pltpu.VMEM((2 * IBK, ECH), jnp.int32),
          pltpu.VMEM((2 * IBK, ECH), jnp.int32),
          pltpu.VMEM((2 * IBK, ECH), jnp.int32),
          pltpu.VMEM((2 * IBK, ECH), jnp.int32),
      ] + [pltpu.VMEM((ECH, HH), jnp.float32)] * NBUF + [
          pltpu.VMEM((ZR, HH), jnp.float32),
          pltpu.VMEM_SHARED((ACCN, HH), jnp.float32),
      ] + [pltpu.SemaphoreType.DMA] * (2 * NBUF),
  )
  deg = pl.kernel(
      _deg_body,
      out_type=jax.ShapeDtypeStruct((2 * ACCN, HH), jnp.float32),
      mesh=mesh,
      scratch_types=[
          pltpu.VMEM((NCH_D, DCH), jnp.int32),
          pltpu.VMEM((DCH, HH), jnp.float32),
          pltpu.VMEM_SHARED((ACCN, HH), jnp.float32),
          pltpu.SemaphoreType.DMA,
      ],
  )
  return agg, deg


def kernel(x, edge_index, lin1_W, lin1_b, blk_g, blk_b, conv_Wl, conv_bl,
           conv_Wr, codebooks, fin_g, fin_b, lin2_W, lin2_b):
  srcp = jnp.concatenate([edge_index[0], jnp.zeros((EP - E,), jnp.int32)])
  dstp = jnp.concatenate([edge_index[1], jnp.full((EP - E,), N, jnp.int32)])
  src2r = jnp.concatenate([srcp, srcp + N]).reshape(2 * EP // ECH, ECH)
  dstr = dstp.reshape(EP // ECH, ECH)
  dstr_d = dstp.reshape(EP // DCH, DCH)

  w1t = lin1_W.T
  w2t = lin2_W.T
  cbp = jnp.pad(codebooks, ((0, 0), (0, 0), (0, KP - K), (0, 0)))
  cbsq = jnp.pad((codebooks * codebooks).sum(-1),
                 ((0, 0), (0, 0), (0, KP - K)),
                 constant_values=1e30)[:, :, None, :]
  zeros128 = jnp.zeros((ACCN, HH), jnp.float32)
  ones128 = jnp.ones((DCH, HH), jnp.float32)

  agg_kernel, deg_kernel = _sc_kernels()
  h2 = _pre_call(x, w1t, lin1_b[None], blk_g[0][None], blk_b[0][None])
  degf = deg_kernel(dstr_d, zeros128, ones128)
  degp = jnp.stack([degf[:N], degf[ACCN:ACCN + N]])

  losses = []
  ids = []
  out = None
  for i in range(L):
    aggf = agg_kernel(h2.reshape(2 * N, HH), src2r, dstr)
    agg = jnp.stack([aggf[:N], aggf[ACCN:ACCN + N]])
    last = i == L - 1
    gn = fin_g[None] if last else blk_g[i + 1][None]
    bn = fin_b[None] if last else blk_b[i + 1][None]
    o, ids_i, lp = _layer_call(
        last, h2, agg, degp,
        conv_Wl[i].T, conv_Wr[i].T, conv_bl[i][None],
        cbp[i], cbsq[i], gn, bn, w2t, lin2_b[None])
    if last:
      out = o
    else:
      h2 = o
    losses.append(jnp.sum(lp[:, 0, 0]) / (N * H))
    ids.append(ids_i[:, :G])

  total_loss = losses[0] + losses[1] + losses[2]
  return out, total_loss, jnp.concatenate(ids, axis=1)
